# Initial kernel scaffold; baseline (speedup 1.0000x reference)
import jax
import jax.numpy as jnp
from jax.experimental import pallas as pl


def kernel(x, edge_index, edge_attr, edge_weight, batch, params):
    raise NotImplementedError("write your pallas kernel here")


# speedup vs baseline: 3.7753x; 3.7753x over previous
import functools

import jax
import jax.numpy as jnp
from jax import lax
from jax.experimental import pallas as pl
from jax.experimental.pallas import tpu as pltpu
from jax.experimental.pallas import tpu_sc as plsc

N = 10000
E = 320000
EMB = 128
N_CLASSES = 12
N_GRAPHS = 64

NC = 2
NS = 16
NW = NC * NS
EPT = E // NW
CHUNK = 80
NCHUNK = EPT // CHUNK
S_ROWS = 624
ZROWS = 104

_NEG_INF = float("-inf")


def _mlp_block(z, W1, b1, g1, be1, W2, b2, g2, be2):
    y = jnp.dot(z, W1, preferred_element_type=jnp.float32) + b1
    m = jnp.mean(y, axis=0, keepdims=True)
    v = jnp.mean((y - m) ** 2, axis=0, keepdims=True)
    h1 = jnp.maximum(g1 * (y - m) / jnp.sqrt(v + 1e-5) + be1, 0.0)
    y2 = jnp.dot(h1, W2, preferred_element_type=jnp.float32) + b2
    m2 = jnp.mean(y2, axis=0, keepdims=True)
    v2 = jnp.mean((y2 - m2) ** 2, axis=0, keepdims=True)
    return jnp.maximum(g2 * (y2 - m2) / jnp.sqrt(v2 + 1e-5) + be2, 0.0)


def _pool_from(h2, WoT, bo, batch_row):
    yT = lax.dot_general(WoT, h2, (((1,), (1,)), ((), ())),
                         preferred_element_type=jnp.float32) + bo
    gids = lax.broadcasted_iota(jnp.int32, (N_GRAPHS, 1), 0)
    mask = batch_row == gids
    cols = []
    for c in range(N_CLASSES):
        vc = jnp.where(mask, yT[c:c + 1, :], _NEG_INF)
        cols.append(jnp.max(vc, axis=1, keepdims=True))
    return jnp.concatenate(cols, axis=1)


def _init_body(x_ref, W1, b1, g1, be1, W2, b2, g2, be2, WoT, bo, batch_ref,
               h_ref, out_ref):
    h2 = _mlp_block(x_ref[...], W1[...], b1[...], g1[...], be1[...],
                    W2[...], b2[...], g2[...], be2[...])
    h_ref[...] = h2
    out_ref[...] = _pool_from(h2, WoT[...], bo[...], batch_ref[...])


def _stage_body(h_ref, aggs_ref, sc_ref, W1, b1, g1, be1, W2, b2, g2, be2,
                WoT, bo, batch_ref, outprev_ref, h_new_ref, out_new_ref):
    z = h_ref[...] * sc_ref[...] + aggs_ref[0:N, :] + aggs_ref[N:2 * N, :]
    h2 = _mlp_block(z, W1[...], b1[...], g1[...], be1[...],
                    W2[...], b2[...], g2[...], be2[...])
    h_new_ref[...] = h2
    out_new_ref[...] = outprev_ref[...] + _pool_from(
        h2, WoT[...], bo[...], batch_ref[...])


_f32 = jnp.float32


def _tc_init(x, w, batch_row):
    return pl.pallas_call(
        _init_body,
        out_shape=(jax.ShapeDtypeStruct((N, EMB), _f32),
                   jax.ShapeDtypeStruct((N_GRAPHS, N_CLASSES), _f32)),
    )(x, *w, batch_row)


def _tc_stage(h, aggs, scale, w, batch_row, out_prev):
    return pl.pallas_call(
        _stage_body,
        out_shape=(jax.ShapeDtypeStruct((N, EMB), _f32),
                   jax.ShapeDtypeStruct((N_GRAPHS, N_CLASSES), _f32)),
    )(h, aggs, scale, *w, batch_row, out_prev)


def _sc_agg_body(h_hbm, src_hbm, dst_hbm, w_hbm, out_hbm,
                 sidx, didx, wv, rows, zbuf, aggsh, sem):
    c = lax.axis_index("c")
    s = lax.axis_index("s")
    wid = c * NS + s

    def zrow(r, carry):
        for j in range(EMB // 16):
            zbuf[r, pl.ds(j * 16, 16)] = jnp.zeros((16,), _f32)
        return carry

    lax.fori_loop(0, ZROWS, zrow, 0)
    for k in range(S_ROWS // ZROWS):
        pltpu.sync_copy(zbuf, aggsh.at[pl.ds(s * S_ROWS + k * ZROWS, ZROWS)])

    @pl.when(s == 0)
    def _zero_tail():
        pltpu.sync_copy(zbuf.at[pl.ds(0, 16)],
                        aggsh.at[pl.ds(NS * S_ROWS, N - NS * S_ROWS)])

    plsc.subcore_barrier()

    def body(i, carry):
        base = wid * EPT + i * CHUNK
        pltpu.sync_copy(src_hbm.at[pl.ds(base, CHUNK)], sidx)
        pltpu.sync_copy(dst_hbm.at[pl.ds(base, CHUNK)], didx)
        pltpu.sync_copy(w_hbm.at[pl.ds(base, CHUNK)], wv)
        pltpu.async_copy(h_hbm.at[sidx], rows, sem).wait()

        def mul(g, cc):
            w16 = wv[pl.ds(g * 16, 16)]
            for e in range(16):
                wgt = w16[e]
                r = g * 16 + e
                for j in range(EMB // 16):
                    rows[r, pl.ds(j * 16, 16)] = rows[r, pl.ds(j * 16, 16)] * wgt
            return cc

        lax.fori_loop(0, CHUNK // 16, mul, 0)
        pltpu.sync_copy(rows, aggsh.at[didx], add=True)
        return carry

    lax.fori_loop(0, NCHUNK, body, 0)
    plsc.subcore_barrier()
    pltpu.sync_copy(aggsh.at[pl.ds(s * S_ROWS, S_ROWS)],
                    out_hbm.at[pl.ds(c * N + s * S_ROWS, S_ROWS)])

    @pl.when(s == 0)
    def _copy_tail():
        pltpu.sync_copy(aggsh.at[pl.ds(NS * S_ROWS, N - NS * S_ROWS)],
                        out_hbm.at[pl.ds(c * N + NS * S_ROWS, N - NS * S_ROWS)])


def _sc_agg(h, src, dst, ew):
    mesh = plsc.VectorSubcoreMesh(core_axis_name="c", subcore_axis_name="s")
    k = pl.kernel(
        _sc_agg_body, mesh=mesh,
        out_type=jax.ShapeDtypeStruct((2 * N, EMB), _f32),
        scratch_types=[
            pltpu.VMEM((CHUNK,), jnp.int32),
            pltpu.VMEM((CHUNK,), jnp.int32),
            pltpu.VMEM((CHUNK,), _f32),
            pltpu.VMEM((CHUNK, EMB), _f32),
            pltpu.VMEM((ZROWS, EMB), _f32),
            pltpu.VMEM_SHARED((N, EMB), _f32),
            pltpu.SemaphoreType.DMA,
        ],
    )
    return k(h, src, dst, ew)


def _stage_weights(mlp, out_lin):
    return (mlp["lin1"]["W"], mlp["lin1"]["b"].reshape(1, EMB),
            mlp["bn1"]["gamma"].reshape(1, EMB), mlp["bn1"]["beta"].reshape(1, EMB),
            mlp["lin2"]["W"], mlp["lin2"]["b"].reshape(1, EMB),
            mlp["bn2"]["gamma"].reshape(1, EMB), mlp["bn2"]["beta"].reshape(1, EMB),
            out_lin["W"].T, out_lin["b"].reshape(N_CLASSES, 1))


def kernel(x, edge_index, edge_attr, edge_weight, batch, params):
    del edge_attr
    src = edge_index[0].astype(jnp.int32)
    dst = edge_index[1].astype(jnp.int32)
    ew = edge_weight.astype(_f32)
    batch_row = batch.astype(jnp.int32).reshape(1, N)

    h, out = _tc_init(x, _stage_weights(params["init_mlp"], params["init_lin"]),
                      batch_row)
    for lp in params["layers"]:
        aggs = _sc_agg(h, src, dst, ew)
        scale = (1.0 + lp["eps"]).astype(_f32).reshape(1, 1)
        h, out = _tc_stage(h, aggs, scale,
                           _stage_weights(lp["mlp"], lp["out_lin"]),
                           batch_row, out)
    return out


# speedup vs baseline: 6.2053x; 1.6437x over previous
import jax
import jax.numpy as jnp
from jax import lax
from jax.experimental import pallas as pl
from jax.experimental.pallas import tpu as pltpu
from jax.experimental.pallas import tpu_sc as plsc

N = 10000
E = 320000
EMB = 128
N_CLASSES = 12
N_GRAPHS = 64

NC = 2
NS = 16
NW = NC * NS
EPT = E // NW
CHUNK = 48
NCHUNK = 209
PADE = NCHUNK * CHUNK - EPT
S_ROWS = 624
NBUF = 2
NPBUF = 6
BLK = 6
MAIN = 204
NBLK = MAIN // BLK

_f32 = jnp.float32


def _mlp_block(z, W1, b1, g1, be1, W2, b2, g2, be2):
    y = jnp.dot(z, W1, preferred_element_type=jnp.float32) + b1
    m = jnp.mean(y, axis=0, keepdims=True)
    v = jnp.mean((y - m) ** 2, axis=0, keepdims=True)
    h1 = jnp.maximum(g1 * (y - m) / jnp.sqrt(v + 1e-5) + be1, 0.0)
    y2 = jnp.dot(h1, W2, preferred_element_type=jnp.float32) + b2
    m2 = jnp.mean(y2, axis=0, keepdims=True)
    v2 = jnp.mean((y2 - m2) ** 2, axis=0, keepdims=True)
    return jnp.maximum(g2 * (y2 - m2) / jnp.sqrt(v2 + 1e-5) + be2, 0.0)


def _pool_from(h2, WoT, bo, batch_row):
    yT = lax.dot_general(WoT, h2, (((1,), (1,)), ((), ())),
                         preferred_element_type=jnp.float32) + bo
    gids = lax.broadcasted_iota(jnp.int32, (N_GRAPHS, 1), 0)
    mask = batch_row == gids
    cols = []
    for c in range(N_CLASSES):
        vc = jnp.where(mask, yT[c:c + 1, :], float("-inf"))
        cols.append(jnp.max(vc, axis=1, keepdims=True))
    return jnp.concatenate(cols, axis=1)


def _init_body(x_ref, W1, b1, g1, be1, W2, b2, g2, be2, WoT, bo, batch_ref,
               h_ref, out_ref):
    h2 = _mlp_block(x_ref[...], W1[...], b1[...], g1[...], be1[...],
                    W2[...], b2[...], g2[...], be2[...])
    h_ref[...] = h2
    out_ref[...] = _pool_from(h2, WoT[...], bo[...], batch_ref[...])


def _stage_body(h_ref, aggs_ref, sc_ref, W1, b1, g1, be1, W2, b2, g2, be2,
                WoT, bo, batch_ref, outprev_ref, h_new_ref, out_new_ref):
    z = h_ref[...] * sc_ref[...] + aggs_ref[0:N, :] + aggs_ref[N:2 * N, :]
    h2 = _mlp_block(z, W1[...], b1[...], g1[...], be1[...],
                    W2[...], b2[...], g2[...], be2[...])
    h_new_ref[...] = h2
    out_new_ref[...] = outprev_ref[...] + _pool_from(
        h2, WoT[...], bo[...], batch_ref[...])


def _tc_init(x, w, batch_row):
    return pl.pallas_call(
        _init_body,
        out_shape=(jax.ShapeDtypeStruct((N, EMB), _f32),
                   jax.ShapeDtypeStruct((N_GRAPHS, N_CLASSES), _f32)),
    )(x, *w, batch_row)


def _tc_stage(h, aggs, scale, w, batch_row, out_prev):
    return pl.pallas_call(
        _stage_body,
        out_shape=(jax.ShapeDtypeStruct((N, EMB), _f32),
                   jax.ShapeDtypeStruct((N_GRAPHS, N_CLASSES), _f32)),
    )(h, aggs, scale, *w, batch_row, out_prev)


def _sc_agg_body(h_hbm, pk_hbm, w_hbm, z_hbm, out_hbm, *refs):
    pbuf = refs[0:NPBUF]
    wbuf = refs[NPBUF:2 * NPBUF]
    rin = refs[2 * NPBUF:2 * NPBUF + NBUF]
    rout = refs[2 * NPBUF + NBUF:2 * NPBUF + 2 * NBUF]
    aggsh = refs[2 * NPBUF + 2 * NBUF]
    _sems = refs[2 * NPBUF + 2 * NBUF + 1:]
    isem = _sems[0:NPBUF]
    gsem = _sems[NPBUF:NPBUF + NBUF]
    ssem = _sems[NPBUF + NBUF:NPBUF + 2 * NBUF]

    c = lax.axis_index("c")
    s = lax.axis_index("s")
    wid = c * NS + s

    pltpu.sync_copy(z_hbm.at[pl.ds(0, S_ROWS)],
                    aggsh.at[pl.ds(s * S_ROWS, S_ROWS)])

    @pl.when(s == 0)
    def _zero_tail():
        pltpu.sync_copy(z_hbm.at[pl.ds(0, N - NS * S_ROWS)],
                        aggsh.at[pl.ds(NS * S_ROWS, N - NS * S_ROWS)])

    plsc.subcore_barrier()

    def p_start(ci, bp):
        pltpu.async_copy(pk_hbm.at[wid * NCHUNK + ci], pbuf[bp], isem[bp])
        pltpu.async_copy(w_hbm.at[wid * NCHUNK + ci], wbuf[bp], isem[bp])

    def p_wait(ci, bp):
        pltpu.make_async_copy(pk_hbm.at[wid * NCHUNK + ci], pbuf[bp],
                              isem[bp]).wait()
        pltpu.make_async_copy(w_hbm.at[wid * NCHUNK + ci], wbuf[bp],
                              isem[bp]).wait()

    def g_start(bp, b2):
        pltpu.async_copy(h_hbm.at[pbuf[bp].at[0]], rin[b2], gsem[b2])

    def g_wait(bp, b2):
        pltpu.make_async_copy(h_hbm.at[pbuf[bp].at[0]], rin[b2],
                              gsem[b2]).wait()

    def s_start(bp, b2):
        pltpu.async_copy(rout[b2], aggsh.at[pbuf[bp].at[1]], ssem[b2],
                         add=True)

    def s_wait(bp, b2):
        pltpu.make_async_copy(rout[b2], aggsh.at[pbuf[bp].at[1]],
                              ssem[b2]).wait()

    def mul(bp, b2):
        def grp(m, cc):
            w16 = wbuf[bp][pl.ds(m * 16, 16)]
            for e in range(16):
                wgt = w16[e]
                r = m * 16 + e
                for j in range(EMB // 16):
                    rout[b2][r, pl.ds(j * 16, 16)] = (
                        rin[b2][r, pl.ds(j * 16, 16)] * wgt)
            return cc

        lax.fori_loop(0, CHUNK // 16, grp, 0)

    def step(ci, b, drain, pstart, gstart):
        b2 = b % NBUF
        g_wait(b, b2)
        if drain is not False:
            dbp = (b - NBUF) % NPBUF
            if drain is True:
                s_wait(dbp, b2)
            else:
                @pl.when(drain)
                def _drain():
                    s_wait(dbp, b2)
        if pstart:
            p_start(ci + 4, (b + 4) % NPBUF)
        mul(b, b2)
        s_start(b, b2)
        if gstart:
            p_wait(ci + NBUF, (b + NBUF) % NPBUF)
            g_start((b + NBUF) % NPBUF, (b + NBUF) % NBUF)

    for i in range(4):
        p_start(i, i)
    for i in range(NBUF):
        p_wait(i, i)
        g_start(i, i)

    def block(g, carry):
        for b in range(BLK):
            drain = True if b >= NBUF else (g > 0)
            step(g * BLK + b, b, drain, True, True)
        return carry

    lax.fori_loop(0, NBLK, block, 0)

    step(MAIN + 0, 0, True, True, True)
    step(MAIN + 1, 1, True, False, True)
    step(MAIN + 2, 2, True, False, True)
    step(MAIN + 3, 3, True, False, False)
    step(MAIN + 4, 4, True, False, False)
    s_wait(3, 1)
    s_wait(4, 0)

    plsc.subcore_barrier()
    pltpu.sync_copy(aggsh.at[pl.ds(s * S_ROWS, S_ROWS)],
                    out_hbm.at[pl.ds(c * N + s * S_ROWS, S_ROWS)])

    @pl.when(s == 0)
    def _copy_tail():
        pltpu.sync_copy(aggsh.at[pl.ds(NS * S_ROWS, N - NS * S_ROWS)],
                        out_hbm.at[pl.ds(c * N + NS * S_ROWS, N - NS * S_ROWS)])


def _sc_agg(h, packed, wts, zrs):
    mesh = plsc.VectorSubcoreMesh(core_axis_name="c", subcore_axis_name="s")
    k = pl.kernel(
        _sc_agg_body, mesh=mesh,
        out_type=jax.ShapeDtypeStruct((2 * N, EMB), _f32),
        scratch_types=(
            [pltpu.VMEM((2, CHUNK), jnp.int32)] * NPBUF
            + [pltpu.VMEM((CHUNK,), _f32)] * NPBUF
            + [pltpu.VMEM((CHUNK, EMB), _f32)] * (2 * NBUF)
            + [pltpu.VMEM_SHARED((N, EMB), _f32)]
            + [pltpu.SemaphoreType.DMA] * (NPBUF + 2 * NBUF)
        ),
    )
    return k(h, packed, wts, zrs)


def _stage_weights(mlp, out_lin):
    return (mlp["lin1"]["W"], mlp["lin1"]["b"].reshape(1, EMB),
            mlp["bn1"]["gamma"].reshape(1, EMB), mlp["bn1"]["beta"].reshape(1, EMB),
            mlp["lin2"]["W"], mlp["lin2"]["b"].reshape(1, EMB),
            mlp["bn2"]["gamma"].reshape(1, EMB), mlp["bn2"]["beta"].reshape(1, EMB),
            out_lin["W"].T, out_lin["b"].reshape(N_CLASSES, 1))


def kernel(x, edge_index, edge_attr, edge_weight, batch, params):
    del edge_attr
    pad2 = ((0, 0), (0, PADE))
    src = jnp.pad(edge_index[0].astype(jnp.int32).reshape(NW, EPT), pad2)
    dst = jnp.pad(edge_index[1].astype(jnp.int32).reshape(NW, EPT), pad2)
    packed = jnp.concatenate(
        [src.reshape(NW * NCHUNK, 1, CHUNK),
         dst.reshape(NW * NCHUNK, 1, CHUNK)], axis=1)
    wts = jnp.pad(edge_weight.astype(_f32).reshape(NW, EPT),
                  pad2).reshape(NW * NCHUNK, CHUNK)
    zrs = jnp.zeros((S_ROWS, EMB), _f32)
    batch_row = batch.astype(jnp.int32).reshape(1, N)

    h, out = _tc_init(x, _stage_weights(params["init_mlp"], params["init_lin"]),
                      batch_row)
    for lp in params["layers"]:
        aggs = _sc_agg(h, packed, wts, zrs)
        scale = (1.0 + lp["eps"]).astype(_f32).reshape(1, 1)
        h, out = _tc_stage(h, aggs, scale,
                           _stage_weights(lp["mlp"], lp["out_lin"]),
                           batch_row, out)
    return out
